# baseline (speedup 1.0000x reference)
import functools

import jax
import jax.numpy as jnp
from jax import lax
from jax.experimental import pallas as pl
from jax.experimental.pallas import tpu as pltpu
from jax.experimental.pallas import tpu_sc as plsc

SEQ = 2048
D = 768
WDIM = 20
WPAD = 32
NWIN = 8
NSPAN = NWIN * SEQ - (NWIN * (NWIN - 1)) // 2
OUTD = 2 * D + WDIM
R = 64
CHUNKS_PER_WORKER = 8


def _build():
    info = plsc.get_sparse_core_info()
    nc = info.num_cores

    mesh = plsc.VectorSubcoreMesh(core_axis_name="c", subcore_axis_name="s")

    @functools.partial(
        pl.kernel,
        mesh=mesh,
        out_type=jax.ShapeDtypeStruct((NSPAN, OUTD), jnp.float32),
        scratch_types=[
            pltpu.VMEM((R, D), jnp.float32),
            pltpu.VMEM((R, WDIM), jnp.float32),
            pltpu.SemaphoreType.DMA,
            pltpu.SemaphoreType.DMA,
        ],
        compiler_params=pltpu.CompilerParams(use_tc_tiling_on_sc=False),
    )
    def k(x_hbm, swe_hbm, out_hbm, buf, wbuf, sem, fill_sem):
        wid = lax.axis_index("s") * nc + lax.axis_index("c")
        wi = wid // 4
        q = wid % 4
        n = SEQ - wi
        off = SEQ * wi - (wi * (wi - 1)) // 2
        b = wi + 1 - (wi >= 5).astype(jnp.int32)

        fills = [
            pltpu.make_async_copy(
                swe_hbm.at[pl.ds(b, 1), :], wbuf.at[pl.ds(r, 1), :], fill_sem
            )
            for r in range(R)
        ]
        for cp in fills:
            cp.start()

        copies = []
        for t in range(CHUNKS_PER_WORKER):
            c = q * CHUNKS_PER_WORKER + t
            j0 = jnp.minimum(c * R, n - R)
            j1 = j0 + wi
            r0 = off + j0
            copies.append(pltpu.make_async_copy(
                x_hbm.at[pl.ds(j0, R), :],
                out_hbm.at[pl.ds(r0, R), pl.ds(0, D)], sem))
            copies.append(pltpu.make_async_copy(
                x_hbm.at[pl.ds(j1, R), :],
                out_hbm.at[pl.ds(r0, R), pl.ds(D, D)], sem))
        for cp in copies:
            cp.start()
        for cp in fills:
            cp.wait()
        for t in range(CHUNKS_PER_WORKER):
            c = q * CHUNKS_PER_WORKER + t
            j0 = jnp.minimum(c * R, n - R)
            r0 = off + j0
            pltpu.sync_copy(
                wbuf, out_hbm.at[pl.ds(r0, R), pl.ds(2 * D, WDIM)]
            )
        for cp in copies:
            cp.wait()

    return k


def kernel(x, span_width_embedding, batch_max_seq_len):
    del batch_max_seq_len
    x2 = x.reshape(SEQ, D)
    out = _build()(x2, span_width_embedding)
    return out.reshape(1, NSPAN, OUTD)


# speedup vs baseline: 5.6173x; 5.6173x over previous
import functools

import jax
import jax.numpy as jnp
from jax import lax
from jax.experimental import pallas as pl
from jax.experimental.pallas import tpu as pltpu
from jax.experimental.pallas import tpu_sc as plsc

SEQ = 2048
D = 768
WDIM = 20
NWIN = 8
NSPAN = NWIN * SEQ - (NWIN * (NWIN - 1)) // 2
OUTD = 2 * D + WDIM
R = 64
CHUNKS_PER_WORKER = 8


def _build():
    info = plsc.get_sparse_core_info()
    nc = info.num_cores

    mesh = plsc.VectorSubcoreMesh(core_axis_name="c", subcore_axis_name="s")

    @functools.partial(
        pl.kernel,
        mesh=mesh,
        out_type=jax.ShapeDtypeStruct((NSPAN, OUTD), jnp.float32),
        scratch_types=[
            pltpu.VMEM_SHARED((SEQ, D), jnp.float32),
            pltpu.VMEM((R, WDIM), jnp.float32),
            pltpu.SemaphoreType.DMA,
            pltpu.SemaphoreType.DMA,
        ],
        compiler_params=pltpu.CompilerParams(use_tc_tiling_on_sc=False),
    )
    def k(x_hbm, swe_hbm, out_hbm, xs, wbuf, sem, fill_sem):
        cid = lax.axis_index("c")
        sid = lax.axis_index("s")
        wid = sid * nc + cid
        wi = wid // 4
        q = wid % 4
        n = SEQ - wi
        off = SEQ * wi - (wi * (wi - 1)) // 2
        b = wi + 1 - (wi >= 5).astype(jnp.int32)

        fills = [
            pltpu.make_async_copy(
                swe_hbm.at[pl.ds(b, 1), :], wbuf.at[pl.ds(r, 1), :], fill_sem
            )
            for r in range(R)
        ]
        for cp in fills:
            cp.start()

        @pl.when(sid == 0)
        def _stage():
            pltpu.sync_copy(x_hbm, xs)

        plsc.subcore_barrier()

        copies = []
        for t in range(CHUNKS_PER_WORKER):
            c = q * CHUNKS_PER_WORKER + t
            j0 = jnp.minimum(c * R, n - R)
            j1 = j0 + wi
            r0 = off + j0
            copies.append(pltpu.make_async_copy(
                xs.at[pl.ds(j0, R), :],
                out_hbm.at[pl.ds(r0, R), pl.ds(0, D)], sem))
            copies.append(pltpu.make_async_copy(
                xs.at[pl.ds(j1, R), :],
                out_hbm.at[pl.ds(r0, R), pl.ds(D, D)], sem))
        for cp in copies:
            cp.start()
        for cp in fills:
            cp.wait()
        for t in range(CHUNKS_PER_WORKER):
            c = q * CHUNKS_PER_WORKER + t
            j0 = jnp.minimum(c * R, n - R)
            r0 = off + j0
            copies.append(pltpu.make_async_copy(
                wbuf, out_hbm.at[pl.ds(r0, R), pl.ds(2 * D, WDIM)], sem))
            copies[-1].start()
        for cp in copies:
            cp.wait()

    return k


def kernel(x, span_width_embedding, batch_max_seq_len):
    del batch_max_seq_len
    x2 = x.reshape(SEQ, D)
    out = _build()(x2, span_width_embedding)
    return out.reshape(1, NSPAN, OUTD)
